# Initial kernel scaffold; baseline (speedup 1.0000x reference)
import jax
import jax.numpy as jnp
from jax.experimental import pallas as pl


def kernel(x, edge_index, W0, b0, g1_wgate, g1_expW, g1_expb, W2, b2, g3_wgate, g3_expW, g3_expb, W4, b4, W5, b5):
    raise NotImplementedError("write your pallas kernel here")


# speedup vs baseline: 20.1719x; 20.1719x over previous
import functools

import jax
import jax.numpy as jnp
from jax import lax
from jax.experimental import pallas as pl
from jax.experimental.pallas import tpu as pltpu
from jax.experimental.pallas import tpu_sc as plsc

_N = 10000
_E = 160000
_D = 256
_H = 128
_NEXP = 8
_NTILE = 16
_CHUNK = 128
_NCHUNK = 80
_EPAD = _NTILE * _NCHUNK * _CHUNK
_ACC_ROWS = 10240
_ZROWS = _ACC_ROWS // _NTILE
_B = 1000
_GRID = _N // _B


def _segsum_body(x0, x1, src_hbm, dst_hbm, zeros_hbm, out0, out1,
                 src_v, dst_v, rows_v, acc, sem):
    c = lax.axis_index("c")
    s = lax.axis_index("s")
    pltpu.sync_copy(src_hbm.at[s], src_v)
    pltpu.sync_copy(dst_hbm.at[s], dst_v)
    pltpu.sync_copy(zeros_hbm, acc.at[pl.ds(s * _ZROWS, _ZROWS)])
    plsc.subcore_barrier()

    def run(x_hbm):
        def body(j, carry):
            pltpu.async_copy(x_hbm.at[src_v.at[j]], rows_v, sem).wait()
            pltpu.sync_copy(rows_v, acc.at[dst_v.at[j]], add=True)
            return carry
        lax.fori_loop(0, _NCHUNK, body, 0)

    @pl.when(c == 0)
    def _():
        run(x0)

    @pl.when(c == 1)
    def _():
        run(x1)

    plsc.subcore_barrier()
    r0 = s * _ZROWS

    @pl.when(c == 0)
    def _():
        pltpu.sync_copy(acc.at[pl.ds(r0, _ZROWS)], out0.at[pl.ds(r0, _ZROWS)])

    @pl.when(c == 1)
    def _():
        pltpu.sync_copy(acc.at[pl.ds(r0, _ZROWS)], out1.at[pl.ds(r0, _ZROWS)])


@functools.cache
def _segsum_call():
    return pl.kernel(
        _segsum_body,
        out_type=(
            jax.ShapeDtypeStruct((_ACC_ROWS, _H), jnp.float32),
            jax.ShapeDtypeStruct((_ACC_ROWS, _H), jnp.float32),
        ),
        mesh=plsc.VectorSubcoreMesh(core_axis_name="c", subcore_axis_name="s",
                                    num_cores=2, num_subcores=_NTILE),
        scratch_types=[
            pltpu.VMEM((_NCHUNK, _CHUNK), jnp.int32),
            pltpu.VMEM((_NCHUNK, _CHUNK), jnp.int32),
            pltpu.VMEM((_CHUNK, _H), jnp.float32),
            pltpu.VMEM_SHARED((_ACC_ROWS, _H), jnp.float32),
            pltpu.SemaphoreType.DMA,
        ],
    )


def _lin_relu_gate_body(a0, a1, w, b, wg, h0, h1, lg):
    h = jnp.dot(a0[...], w[:_H, :], preferred_element_type=jnp.float32)
    h = h + jnp.dot(a1[...], w[_H:, :], preferred_element_type=jnp.float32)
    h = jnp.maximum(h + b[...], 0.0)
    h0[...] = h[:, :_H]
    h1[...] = h[:, _H:]
    lg[...] = jnp.dot(h, wg[...], preferred_element_type=jnp.float32)


def _lin_relu_body(a0, a1, w, b, h0, h1):
    h = jnp.dot(a0[...], w[:_H, :], preferred_element_type=jnp.float32)
    h = h + jnp.dot(a1[...], w[_H:, :], preferred_element_type=jnp.float32)
    h = jnp.maximum(h + b[...], 0.0)
    h0[...] = h[:, :_H]
    h1[...] = h[:, _H:]


def _lin_final_body(a0, a1, w, b, out):
    h = jnp.dot(a0[...], w[:_H, :], preferred_element_type=jnp.float32)
    h = h + jnp.dot(a1[...], w[_H:, :], preferred_element_type=jnp.float32)
    out[...] = h + b[...]


def _top2_gates(lgv):
    eid = lax.broadcasted_iota(jnp.int32, lgv.shape, 1)
    m1 = jnp.max(lgv, axis=1, keepdims=True)
    i1 = jnp.min(jnp.where(lgv == m1, eid, _NEXP), axis=1, keepdims=True)
    mask1 = eid == i1
    lg2 = jnp.where(mask1, -jnp.inf, lgv)
    m2 = jnp.max(lg2, axis=1, keepdims=True)
    i2 = jnp.min(jnp.where(lg2 == m2, eid, _NEXP), axis=1, keepdims=True)
    mask2 = eid == i2
    w2 = jnp.exp(m2 - m1)
    denom = 1.0 + w2
    return jnp.where(mask1, 1.0 / denom, 0.0) + jnp.where(mask2, w2 / denom, 0.0)


def _moe_body(a0, a1, lg, expw, expb, h0, h1):
    gates = _top2_gates(lg[...])
    av0 = a0[...]
    av1 = a1[...]
    acc = jnp.dot(gates, expb[...], preferred_element_type=jnp.float32)
    for e in range(_NEXP):
        he = jnp.dot(av0, expw[e, :_H, :], preferred_element_type=jnp.float32)
        he = he + jnp.dot(av1, expw[e, _H:, :], preferred_element_type=jnp.float32)
        acc = acc + gates[:, e:e + 1] * he
    acc = jnp.maximum(acc, 0.0)
    h0[...] = acc[:, :_H]
    h1[...] = acc[:, _H:]


_half_spec = pl.BlockSpec((_B, _H), lambda i: (i, 0))
_w_spec = pl.BlockSpec((_D, _D), lambda i: (0, 0))
_b_spec = pl.BlockSpec((1, _D), lambda i: (0, 0))
_wg_spec = pl.BlockSpec((_D, _NEXP), lambda i: (0, 0))
_lg_spec = pl.BlockSpec((_B, _NEXP), lambda i: (i, 0))
_half_shape = jax.ShapeDtypeStruct((_N, _H), jnp.float32)

_lin_relu_gate_call = pl.pallas_call(
    _lin_relu_gate_body,
    grid=(_GRID,),
    in_specs=[_half_spec, _half_spec, _w_spec, _b_spec, _wg_spec],
    out_specs=[_half_spec, _half_spec, _lg_spec],
    out_shape=[_half_shape, _half_shape,
               jax.ShapeDtypeStruct((_N, _NEXP), jnp.float32)],
)

_lin_relu_call = pl.pallas_call(
    _lin_relu_body,
    grid=(_GRID,),
    in_specs=[_half_spec, _half_spec, _w_spec, _b_spec],
    out_specs=[_half_spec, _half_spec],
    out_shape=[_half_shape, _half_shape],
)

_lin_final_call = pl.pallas_call(
    _lin_final_body,
    grid=(_GRID,),
    in_specs=[_half_spec, _half_spec, _w_spec, _b_spec],
    out_specs=pl.BlockSpec((_B, _D), lambda i: (i, 0)),
    out_shape=jax.ShapeDtypeStruct((_N, _D), jnp.float32),
)

_moe_call = pl.pallas_call(
    _moe_body,
    grid=(_GRID,),
    in_specs=[_half_spec, _half_spec, _lg_spec,
              pl.BlockSpec((_NEXP, _D, _D), lambda i: (0, 0, 0)),
              pl.BlockSpec((_NEXP, _D), lambda i: (0, 0))],
    out_specs=[_half_spec, _half_spec],
    out_shape=[_half_shape, _half_shape],
)


def kernel(x, edge_index, W0, b0, g1_wgate, g1_expW, g1_expb,
           W2, b2, g3_wgate, g3_expW, g3_expb, W4, b4, W5, b5):
    ei = edge_index.astype(jnp.int32)
    npad = _EPAD - _E
    src = jnp.concatenate([ei[0], jnp.zeros((npad,), jnp.int32)])
    dst = jnp.concatenate(
        [ei[1], _N + (jnp.arange(npad, dtype=jnp.int32) % (_ACC_ROWS - _N))])
    src3 = src.reshape(_NTILE, _NCHUNK, _CHUNK)
    dst3 = dst.reshape(_NTILE, _NCHUNK, _CHUNK)
    zeros = jnp.zeros((_ZROWS, _H), jnp.float32)

    def segsum(h0, h1):
        return _segsum_call()(h0, h1, src3, dst3, zeros)

    a0, a1 = segsum(x[:, :_H], x[:, _H:])
    h0, h1, lg1 = _lin_relu_gate_call(a0, a1, W0, b0.reshape(1, _D), g1_wgate)
    a0, a1 = segsum(h0, h1)
    h0, h1 = _moe_call(a0, a1, lg1, g1_expW, g1_expb)
    a0, a1 = segsum(h0, h1)
    h0, h1, lg3 = _lin_relu_gate_call(a0, a1, W2, b2.reshape(1, _D), g3_wgate)
    a0, a1 = segsum(h0, h1)
    h0, h1 = _moe_call(a0, a1, lg3, g3_expW, g3_expb)
    a0, a1 = segsum(h0, h1)
    h0, h1 = _lin_relu_call(a0, a1, W4, b4.reshape(1, _D))
    a0, a1 = segsum(h0, h1)
    return _lin_final_call(a0, a1, W5, b5.reshape(1, _D))


# speedup vs baseline: 23.8167x; 1.1807x over previous
import functools

import jax
import jax.numpy as jnp
from jax import lax
from jax.experimental import pallas as pl
from jax.experimental.pallas import tpu as pltpu
from jax.experimental.pallas import tpu_sc as plsc

_N = 10000
_E = 160000
_D = 256
_H = 128
_NEXP = 8
_NTILE = 16
_CHUNK = 128
_NCHUNK = 80
_EPAD = _NTILE * _NCHUNK * _CHUNK
_ACC_ROWS = 10240
_ZROWS = _ACC_ROWS // _NTILE
_B = 1000
_GRID = _N // _B


def _segsum_body(x0, x1, eidx_hbm, zeros_hbm, out0, out1,
                 idx0_v, idx1_v, rows0_v, rows1_v, acc, sem0, sem1):
    c = lax.axis_index("c")
    s = lax.axis_index("s")
    pltpu.sync_copy(zeros_hbm, acc.at[pl.ds(s * _ZROWS, _ZROWS)])
    plsc.subcore_barrier()

    def run(x_hbm):
        bufs = ((idx0_v, rows0_v, sem0), (idx1_v, rows1_v, sem1))
        for b, (idx, buf, sem) in enumerate(bufs):
            pltpu.sync_copy(eidx_hbm.at[s, b], idx)
            pltpu.async_copy(x_hbm.at[idx.at[0]], buf, sem)

        def body(g, carry):
            for b, (idx, buf, sem) in enumerate(bufs):
                j = g * 2 + b
                pltpu.make_async_copy(x_hbm.at[idx.at[0]], buf, sem).wait()
                pltpu.sync_copy(buf, acc.at[idx.at[1]], add=True)
                nxt = j + 2

                @pl.when(nxt < _NCHUNK)
                def _():
                    pltpu.sync_copy(eidx_hbm.at[s, nxt], idx)
                    pltpu.async_copy(x_hbm.at[idx.at[0]], buf, sem)
            return carry
        lax.fori_loop(0, _NCHUNK // 2, body, 0)

    @pl.when(c == 0)
    def _():
        run(x0)

    @pl.when(c == 1)
    def _():
        run(x1)

    plsc.subcore_barrier()
    r0 = s * _ZROWS

    @pl.when(c == 0)
    def _():
        pltpu.sync_copy(acc.at[pl.ds(r0, _ZROWS)], out0.at[pl.ds(r0, _ZROWS)])

    @pl.when(c == 1)
    def _():
        pltpu.sync_copy(acc.at[pl.ds(r0, _ZROWS)], out1.at[pl.ds(r0, _ZROWS)])


@functools.cache
def _segsum_call():
    return pl.kernel(
        _segsum_body,
        out_type=(
            jax.ShapeDtypeStruct((_ACC_ROWS, _H), jnp.float32),
            jax.ShapeDtypeStruct((_ACC_ROWS, _H), jnp.float32),
        ),
        mesh=plsc.VectorSubcoreMesh(core_axis_name="c", subcore_axis_name="s",
                                    num_cores=2, num_subcores=_NTILE),
        scratch_types=[
            pltpu.VMEM((2, _CHUNK), jnp.int32),
            pltpu.VMEM((2, _CHUNK), jnp.int32),
            pltpu.VMEM((_CHUNK, _H), jnp.float32),
            pltpu.VMEM((_CHUNK, _H), jnp.float32),
            pltpu.VMEM_SHARED((_ACC_ROWS, _H), jnp.float32),
            pltpu.SemaphoreType.DMA,
            pltpu.SemaphoreType.DMA,
        ],
    )


def _lin_relu_gate_body(a0, a1, w, b, wg, h0, h1, lg):
    h = jnp.dot(a0[...], w[:_H, :], preferred_element_type=jnp.float32)
    h = h + jnp.dot(a1[...], w[_H:, :], preferred_element_type=jnp.float32)
    h = jnp.maximum(h + b[...], 0.0)
    h0[...] = h[:, :_H]
    h1[...] = h[:, _H:]
    lg[...] = jnp.dot(h, wg[...], preferred_element_type=jnp.float32)


def _lin_relu_body(a0, a1, w, b, h0, h1):
    h = jnp.dot(a0[...], w[:_H, :], preferred_element_type=jnp.float32)
    h = h + jnp.dot(a1[...], w[_H:, :], preferred_element_type=jnp.float32)
    h = jnp.maximum(h + b[...], 0.0)
    h0[...] = h[:, :_H]
    h1[...] = h[:, _H:]


def _lin_final_body(a0, a1, w, b, out):
    h = jnp.dot(a0[...], w[:_H, :], preferred_element_type=jnp.float32)
    h = h + jnp.dot(a1[...], w[_H:, :], preferred_element_type=jnp.float32)
    out[...] = h + b[...]


def _top2_gates(lgv):
    eid = lax.broadcasted_iota(jnp.int32, lgv.shape, 1)
    m1 = jnp.max(lgv, axis=1, keepdims=True)
    i1 = jnp.min(jnp.where(lgv == m1, eid, _NEXP), axis=1, keepdims=True)
    mask1 = eid == i1
    lg2 = jnp.where(mask1, -jnp.inf, lgv)
    m2 = jnp.max(lg2, axis=1, keepdims=True)
    i2 = jnp.min(jnp.where(lg2 == m2, eid, _NEXP), axis=1, keepdims=True)
    mask2 = eid == i2
    w2 = jnp.exp(m2 - m1)
    denom = 1.0 + w2
    return jnp.where(mask1, 1.0 / denom, 0.0) + jnp.where(mask2, w2 / denom, 0.0)


def _moe_body(a0, a1, lg, expw, expb, h0, h1):
    gates = _top2_gates(lg[...])
    av0 = a0[...]
    av1 = a1[...]
    acc = jnp.dot(gates, expb[...], preferred_element_type=jnp.float32)
    for e in range(_NEXP):
        he = jnp.dot(av0, expw[e, :_H, :], preferred_element_type=jnp.float32)
        he = he + jnp.dot(av1, expw[e, _H:, :], preferred_element_type=jnp.float32)
        acc = acc + gates[:, e:e + 1] * he
    acc = jnp.maximum(acc, 0.0)
    h0[...] = acc[:, :_H]
    h1[...] = acc[:, _H:]


_half_spec = pl.BlockSpec((_B, _H), lambda i: (i, 0))
_w_spec = pl.BlockSpec((_D, _D), lambda i: (0, 0))
_b_spec = pl.BlockSpec((1, _D), lambda i: (0, 0))
_wg_spec = pl.BlockSpec((_D, _NEXP), lambda i: (0, 0))
_lg_spec = pl.BlockSpec((_B, _NEXP), lambda i: (i, 0))
_half_shape = jax.ShapeDtypeStruct((_N, _H), jnp.float32)

_lin_relu_gate_call = pl.pallas_call(
    _lin_relu_gate_body,
    grid=(_GRID,),
    in_specs=[_half_spec, _half_spec, _w_spec, _b_spec, _wg_spec],
    out_specs=[_half_spec, _half_spec, _lg_spec],
    out_shape=[_half_shape, _half_shape,
               jax.ShapeDtypeStruct((_N, _NEXP), jnp.float32)],
)

_lin_relu_call = pl.pallas_call(
    _lin_relu_body,
    grid=(_GRID,),
    in_specs=[_half_spec, _half_spec, _w_spec, _b_spec],
    out_specs=[_half_spec, _half_spec],
    out_shape=[_half_shape, _half_shape],
)

_lin_final_call = pl.pallas_call(
    _lin_final_body,
    grid=(_GRID,),
    in_specs=[_half_spec, _half_spec, _w_spec, _b_spec],
    out_specs=pl.BlockSpec((_B, _D), lambda i: (i, 0)),
    out_shape=jax.ShapeDtypeStruct((_N, _D), jnp.float32),
)

_moe_call = pl.pallas_call(
    _moe_body,
    grid=(_GRID,),
    in_specs=[_half_spec, _half_spec, _lg_spec,
              pl.BlockSpec((_NEXP, _D, _D), lambda i: (0, 0, 0)),
              pl.BlockSpec((_NEXP, _D), lambda i: (0, 0))],
    out_specs=[_half_spec, _half_spec],
    out_shape=[_half_shape, _half_shape],
)


def kernel(x, edge_index, W0, b0, g1_wgate, g1_expW, g1_expb,
           W2, b2, g3_wgate, g3_expW, g3_expb, W4, b4, W5, b5):
    ei = edge_index.astype(jnp.int32)
    npad = _EPAD - _E
    src = jnp.concatenate([ei[0], jnp.zeros((npad,), jnp.int32)])
    dst = jnp.concatenate(
        [ei[1], _N + (jnp.arange(npad, dtype=jnp.int32) % (_ACC_ROWS - _N))])
    eidx = jnp.stack([src.reshape(_NTILE, _NCHUNK, _CHUNK),
                      dst.reshape(_NTILE, _NCHUNK, _CHUNK)], axis=2)
    zeros = jnp.zeros((_ZROWS, _H), jnp.float32)

    def segsum(h0, h1):
        return _segsum_call()(h0, h1, eidx, zeros)

    a0, a1 = segsum(x[:, :_H], x[:, _H:])
    h0, h1, lg1 = _lin_relu_gate_call(a0, a1, W0, b0.reshape(1, _D), g1_wgate)
    a0, a1 = segsum(h0, h1)
    h0, h1 = _moe_call(a0, a1, lg1, g1_expW, g1_expb)
    a0, a1 = segsum(h0, h1)
    h0, h1, lg3 = _lin_relu_gate_call(a0, a1, W2, b2.reshape(1, _D), g3_wgate)
    a0, a1 = segsum(h0, h1)
    h0, h1 = _moe_call(a0, a1, lg3, g3_expW, g3_expb)
    a0, a1 = segsum(h0, h1)
    h0, h1 = _lin_relu_call(a0, a1, W4, b4.reshape(1, _D))
    a0, a1 = segsum(h0, h1)
    return _lin_final_call(a0, a1, W5, b5.reshape(1, _D))


# speedup vs baseline: 24.6483x; 1.0349x over previous
import functools

import jax
import jax.numpy as jnp
from jax import lax
from jax.experimental import pallas as pl
from jax.experimental.pallas import tpu as pltpu
from jax.experimental.pallas import tpu_sc as plsc

_N = 10000
_E = 160000
_D = 256
_H = 128
_NEXP = 8
_NTILE = 16
_CHUNK = 128
_NCHUNK = 80
_EPAD = _NTILE * _NCHUNK * _CHUNK
_ACC_ROWS = 10240
_ZROWS = _ACC_ROWS // _NTILE
_B = 1000
_GRID = _N // _B


def _segsum_body(x0, x1, eidx_hbm, zeros_hbm, out0, out1,
                 idx0_v, idx1_v, idx2_v, idx3_v, rows0_v, rows1_v, acc,
                 isem0, isem1, isem2, isem3, sem0, sem1):
    c = lax.axis_index("c")
    s = lax.axis_index("s")
    pltpu.sync_copy(zeros_hbm, acc.at[pl.ds(s * _ZROWS, _ZROWS)])
    plsc.subcore_barrier()

    def run(x_hbm):
        ibufs = ((idx0_v, isem0), (idx1_v, isem1),
                 (idx2_v, isem2), (idx3_v, isem3))
        dbufs = ((rows0_v, sem0), (rows1_v, sem1))
        for k, (idx, isem) in enumerate(ibufs):
            pltpu.async_copy(eidx_hbm.at[s, k], idx, isem)
        for k in range(2):
            idx, isem = ibufs[k]
            buf, sem = dbufs[k]
            pltpu.make_async_copy(eidx_hbm.at[s, k], idx, isem).wait()
            pltpu.async_copy(x_hbm.at[idx.at[0]], buf, sem)

        def body(g, carry):
            for k in range(4):
                j = g * 4 + k
                idx, _ = ibufs[k]
                buf, sem = dbufs[k % 2]
                pltpu.make_async_copy(x_hbm.at[idx.at[0]], buf, sem).wait()
                pltpu.sync_copy(buf, acc.at[idx.at[1]], add=True)
                nidx, nisem = ibufs[(k + 2) % 4]
                nxt = j + 2

                @pl.when(nxt < _NCHUNK)
                def _():
                    pltpu.make_async_copy(
                        eidx_hbm.at[s, nxt], nidx, nisem).wait()
                    pltpu.async_copy(x_hbm.at[nidx.at[0]], buf, sem)

                pre = j + 4

                @pl.when(pre < _NCHUNK)
                def _():
                    pltpu.async_copy(eidx_hbm.at[s, pre], idx, ibufs[k][1])
            return carry
        lax.fori_loop(0, _NCHUNK // 4, body, 0)

    @pl.when(c == 0)
    def _():
        run(x0)

    @pl.when(c == 1)
    def _():
        run(x1)

    plsc.subcore_barrier()
    r0 = s * _ZROWS

    @pl.when(c == 0)
    def _():
        pltpu.sync_copy(acc.at[pl.ds(r0, _ZROWS)], out0.at[pl.ds(r0, _ZROWS)])

    @pl.when(c == 1)
    def _():
        pltpu.sync_copy(acc.at[pl.ds(r0, _ZROWS)], out1.at[pl.ds(r0, _ZROWS)])


@functools.cache
def _segsum_call():
    return pl.kernel(
        _segsum_body,
        out_type=(
            jax.ShapeDtypeStruct((_ACC_ROWS, _H), jnp.float32),
            jax.ShapeDtypeStruct((_ACC_ROWS, _H), jnp.float32),
        ),
        mesh=plsc.VectorSubcoreMesh(core_axis_name="c", subcore_axis_name="s",
                                    num_cores=2, num_subcores=_NTILE),
        scratch_types=[
            pltpu.VMEM((2, _CHUNK), jnp.int32),
            pltpu.VMEM((2, _CHUNK), jnp.int32),
            pltpu.VMEM((2, _CHUNK), jnp.int32),
            pltpu.VMEM((2, _CHUNK), jnp.int32),
            pltpu.VMEM((_CHUNK, _H), jnp.float32),
            pltpu.VMEM((_CHUNK, _H), jnp.float32),
            pltpu.VMEM_SHARED((_ACC_ROWS, _H), jnp.float32),
            pltpu.SemaphoreType.DMA,
            pltpu.SemaphoreType.DMA,
            pltpu.SemaphoreType.DMA,
            pltpu.SemaphoreType.DMA,
            pltpu.SemaphoreType.DMA,
            pltpu.SemaphoreType.DMA,
        ],
    )


def _lin_relu_gate_body(a0, a1, w, b, wg, h0, h1, lg):
    h = jnp.dot(a0[...], w[:_H, :], preferred_element_type=jnp.float32)
    h = h + jnp.dot(a1[...], w[_H:, :], preferred_element_type=jnp.float32)
    h = jnp.maximum(h + b[...], 0.0)
    h0[...] = h[:, :_H]
    h1[...] = h[:, _H:]
    lg[...] = jnp.dot(h, wg[...], preferred_element_type=jnp.float32)


def _lin_relu_body(a0, a1, w, b, h0, h1):
    h = jnp.dot(a0[...], w[:_H, :], preferred_element_type=jnp.float32)
    h = h + jnp.dot(a1[...], w[_H:, :], preferred_element_type=jnp.float32)
    h = jnp.maximum(h + b[...], 0.0)
    h0[...] = h[:, :_H]
    h1[...] = h[:, _H:]


def _lin_final_body(a0, a1, w, b, out):
    h = jnp.dot(a0[...], w[:_H, :], preferred_element_type=jnp.float32)
    h = h + jnp.dot(a1[...], w[_H:, :], preferred_element_type=jnp.float32)
    out[...] = h + b[...]


def _top2_gates(lgv):
    eid = lax.broadcasted_iota(jnp.int32, lgv.shape, 1)
    m1 = jnp.max(lgv, axis=1, keepdims=True)
    i1 = jnp.min(jnp.where(lgv == m1, eid, _NEXP), axis=1, keepdims=True)
    mask1 = eid == i1
    lg2 = jnp.where(mask1, -jnp.inf, lgv)
    m2 = jnp.max(lg2, axis=1, keepdims=True)
    i2 = jnp.min(jnp.where(lg2 == m2, eid, _NEXP), axis=1, keepdims=True)
    mask2 = eid == i2
    w2 = jnp.exp(m2 - m1)
    denom = 1.0 + w2
    return jnp.where(mask1, 1.0 / denom, 0.0) + jnp.where(mask2, w2 / denom, 0.0)


def _moe_body(a0, a1, lg, expw, expb, h0, h1):
    gates = _top2_gates(lg[...])
    av0 = a0[...]
    av1 = a1[...]
    acc = jnp.dot(gates, expb[...], preferred_element_type=jnp.float32)
    for e in range(_NEXP):
        he = jnp.dot(av0, expw[e, :_H, :], preferred_element_type=jnp.float32)
        he = he + jnp.dot(av1, expw[e, _H:, :], preferred_element_type=jnp.float32)
        acc = acc + gates[:, e:e + 1] * he
    acc = jnp.maximum(acc, 0.0)
    h0[...] = acc[:, :_H]
    h1[...] = acc[:, _H:]


_half_spec = pl.BlockSpec((_B, _H), lambda i: (i, 0))
_w_spec = pl.BlockSpec((_D, _D), lambda i: (0, 0))
_b_spec = pl.BlockSpec((1, _D), lambda i: (0, 0))
_wg_spec = pl.BlockSpec((_D, _NEXP), lambda i: (0, 0))
_lg_spec = pl.BlockSpec((_B, _NEXP), lambda i: (i, 0))
_half_shape = jax.ShapeDtypeStruct((_N, _H), jnp.float32)

_lin_relu_gate_call = pl.pallas_call(
    _lin_relu_gate_body,
    grid=(_GRID,),
    in_specs=[_half_spec, _half_spec, _w_spec, _b_spec, _wg_spec],
    out_specs=[_half_spec, _half_spec, _lg_spec],
    out_shape=[_half_shape, _half_shape,
               jax.ShapeDtypeStruct((_N, _NEXP), jnp.float32)],
)

_lin_relu_call = pl.pallas_call(
    _lin_relu_body,
    grid=(_GRID,),
    in_specs=[_half_spec, _half_spec, _w_spec, _b_spec],
    out_specs=[_half_spec, _half_spec],
    out_shape=[_half_shape, _half_shape],
)

_lin_final_call = pl.pallas_call(
    _lin_final_body,
    grid=(_GRID,),
    in_specs=[_half_spec, _half_spec, _w_spec, _b_spec],
    out_specs=pl.BlockSpec((_B, _D), lambda i: (i, 0)),
    out_shape=jax.ShapeDtypeStruct((_N, _D), jnp.float32),
)

_moe_call = pl.pallas_call(
    _moe_body,
    grid=(_GRID,),
    in_specs=[_half_spec, _half_spec, _lg_spec,
              pl.BlockSpec((_NEXP, _D, _D), lambda i: (0, 0, 0)),
              pl.BlockSpec((_NEXP, _D), lambda i: (0, 0))],
    out_specs=[_half_spec, _half_spec],
    out_shape=[_half_shape, _half_shape],
)


def kernel(x, edge_index, W0, b0, g1_wgate, g1_expW, g1_expb,
           W2, b2, g3_wgate, g3_expW, g3_expb, W4, b4, W5, b5):
    ei = edge_index.astype(jnp.int32)
    npad = _EPAD - _E
    src = jnp.concatenate([ei[0], jnp.zeros((npad,), jnp.int32)])
    dst = jnp.concatenate(
        [ei[1], _N + (jnp.arange(npad, dtype=jnp.int32) % (_ACC_ROWS - _N))])
    eidx = jnp.stack([src.reshape(_NTILE, _NCHUNK, _CHUNK),
                      dst.reshape(_NTILE, _NCHUNK, _CHUNK)], axis=2)
    zeros = jnp.zeros((_ZROWS, _H), jnp.float32)

    def segsum(h0, h1):
        return _segsum_call()(h0, h1, eidx, zeros)

    a0, a1 = segsum(x[:, :_H], x[:, _H:])
    h0, h1, lg1 = _lin_relu_gate_call(a0, a1, W0, b0.reshape(1, _D), g1_wgate)
    a0, a1 = segsum(h0, h1)
    h0, h1 = _moe_call(a0, a1, lg1, g1_expW, g1_expb)
    a0, a1 = segsum(h0, h1)
    h0, h1, lg3 = _lin_relu_gate_call(a0, a1, W2, b2.reshape(1, _D), g3_wgate)
    a0, a1 = segsum(h0, h1)
    h0, h1 = _moe_call(a0, a1, lg3, g3_expW, g3_expb)
    a0, a1 = segsum(h0, h1)
    h0, h1 = _lin_relu_call(a0, a1, W4, b4.reshape(1, _D))
    a0, a1 = segsum(h0, h1)
    return _lin_final_call(a0, a1, W5, b5.reshape(1, _D))
